# Initial kernel scaffold; baseline (speedup 1.0000x reference)
import jax
import jax.numpy as jnp
from jax.experimental import pallas as pl


def kernel(textual_tokens, visual_tokens, text_table, vision_table):
    raise NotImplementedError("write your pallas kernel here")


# speedup vs baseline: 5.1395x; 5.1395x over previous
import functools

import jax
import jax.numpy as jnp
from jax import lax
from jax.experimental import pallas as pl
from jax.experimental.pallas import tpu as pltpu
from jax.experimental.pallas import tpu_sc as plsc

_TEXT_VOCAB = 100000
_VISION_VOCAB = 8192
_D = 64
_BATCH = 4096
_TEXT_LEN = 200
_VIS_LEN = 196

_SEQ = _VIS_LEN + _TEXT_LEN
_NROWS = _BATCH * _SEQ
_NW = 32
_ROWS_PER_W = _NROWS // _NW
_CH = 128
_NCHUNK = _ROWS_PER_W // _CH
_NBUF = 2


@functools.partial(
    pl.kernel,
    mesh=plsc.VectorSubcoreMesh(core_axis_name="c", subcore_axis_name="s"),
    out_type=jax.ShapeDtypeStruct((_NROWS, _D), jnp.float32),
    compiler_params=pltpu.CompilerParams(use_tc_tiling_on_sc=False),
    scratch_types=[
        pltpu.VMEM((_NCHUNK, _CH), jnp.int32),
        pltpu.VMEM((_CH, _D), jnp.float32),
        pltpu.VMEM((_CH, _D), jnp.float32),
        pltpu.SemaphoreType.DMA,
        pltpu.SemaphoreType.DMA,
    ],
)
def _gather_kernel(table, idx, out, idx_v, buf0, buf1, gsem0, gsem1):
    wid = lax.axis_index("s") * 2 + lax.axis_index("c")
    out_base = wid * _ROWS_PER_W

    pltpu.sync_copy(idx.at[wid], idx_v)

    bufs = (buf0, buf1)
    sems = (gsem0, gsem1)

    pltpu.make_async_copy(table.at[idx_v.at[0]], bufs[0], sems[0]).start()

    def body(gg, _):
        for b in range(_NBUF):
            g = gg * _NBUF + b
            nxt = g + 1

            @pl.when(nxt < _NCHUNK)
            def _start():
                pltpu.make_async_copy(
                    table.at[idx_v.at[nxt]], bufs[1 - b], sems[1 - b]
                ).start()

            pltpu.make_async_copy(table.at[idx_v.at[g]], bufs[b], sems[b]).wait()
            pltpu.sync_copy(bufs[b], out.at[pl.ds(out_base + g * _CH, _CH)])
        return 0

    lax.fori_loop(0, _NCHUNK // _NBUF, body, 0)


def kernel(textual_tokens, visual_tokens, text_table, vision_table):
    idx = jnp.concatenate(
        [
            visual_tokens.astype(jnp.int32),
            textual_tokens.astype(jnp.int32) + _VISION_VOCAB,
        ],
        axis=1,
    ).reshape(_NW, _NCHUNK, _CH)
    table = jnp.concatenate([vision_table, text_table], axis=0)
    out = _gather_kernel(table, idx)
    return out.reshape(_BATCH, _SEQ, _D)


# speedup vs baseline: 5.3234x; 1.0358x over previous
import functools

import jax
import jax.numpy as jnp
from jax import lax
from jax.experimental import pallas as pl
from jax.experimental.pallas import tpu as pltpu
from jax.experimental.pallas import tpu_sc as plsc

_TEXT_VOCAB = 100000
_VISION_VOCAB = 8192
_D = 64
_BATCH = 4096
_TEXT_LEN = 200
_VIS_LEN = 196

_SEQ = _VIS_LEN + _TEXT_LEN
_NROWS = _BATCH * _SEQ
_NW = 32
_ROWS_PER_W = _NROWS // _NW
_CH = 128
_NCHUNK = _ROWS_PER_W // _CH
_NBUF = 4
_F = 2


@functools.partial(
    pl.kernel,
    mesh=plsc.VectorSubcoreMesh(core_axis_name="c", subcore_axis_name="s"),
    out_type=jax.ShapeDtypeStruct((_NROWS, _D), jnp.float32),
    compiler_params=pltpu.CompilerParams(use_tc_tiling_on_sc=False),
    scratch_types=[
        pltpu.VMEM((_NCHUNK, _CH), jnp.int32),
        pltpu.VMEM((_NBUF, _CH, _D), jnp.float32),
    ]
    + [pltpu.SemaphoreType.DMA] * (2 * _NBUF),
)
def _gather_kernel(table, idx, out, idx_v, ring, *sems):
    gsems = sems[:_NBUF]
    wsems = sems[_NBUF:]
    wid = lax.axis_index("s") * 2 + lax.axis_index("c")
    out_base = wid * _ROWS_PER_W

    pltpu.sync_copy(idx.at[wid], idx_v)

    def gather(c, s):
        return pltpu.make_async_copy(table.at[idx_v.at[c]], ring.at[s], gsems[s])

    def write(c, s):
        return pltpu.make_async_copy(
            ring.at[s], out.at[pl.ds(out_base + c * _CH, _CH)], wsems[s]
        )

    for c in range(_F):
        gather(c, c).start()

    def body(gg, _):
        for b in range(_NBUF):
            g = gg * _NBUF + b
            s = b
            pf = g + _F
            spf = (b + _F) % _NBUF

            @pl.when(pf < _NCHUNK)
            def _prefetch():
                @pl.when(pf >= _NBUF)
                def _drain_prev_write():
                    write(pf - _NBUF, spf).wait()

                gather(pf, spf).start()

            gather(g, s).wait()
            write(g, s).start()
        return 0

    lax.fori_loop(0, _NCHUNK // _NBUF, body, 0)

    for b in range(_NBUF):
        write(_NCHUNK - _NBUF + b, b).wait()


def kernel(textual_tokens, visual_tokens, text_table, vision_table):
    idx = jnp.concatenate(
        [
            visual_tokens.astype(jnp.int32),
            textual_tokens.astype(jnp.int32) + _VISION_VOCAB,
        ],
        axis=1,
    ).reshape(_NW, _NCHUNK, _CH)
    table = jnp.concatenate([vision_table, text_table], axis=0)
    out = _gather_kernel(table, idx)
    return out.reshape(_BATCH, _SEQ, _D)
